# Initial kernel scaffold; baseline (speedup 1.0000x reference)
import jax
import jax.numpy as jnp
from jax.experimental import pallas as pl


def kernel(hidden_states, w13, w2, gate, shared_w1, shared_w2, shared_gate_w):
    raise NotImplementedError("write your pallas kernel here")


# speedup vs baseline: 1.0183x; 1.0183x over previous
import functools

import jax
import jax.numpy as jnp
from jax.experimental import pallas as pl
from jax.experimental.pallas import tpu as pltpu


def _ffn_body(x_ref, wg_ref, wu_ref, ws_ref, w2_ref, sw2_ref, sgw_ref,
              out_ref, *, j_main):
    j = pl.program_id(1)
    x = x_ref[...]

    @pl.when(j == 0)
    def _init():
        out_ref[...] = jnp.zeros_like(out_ref)

    dn = (((1,), (1,)), ((), ()))

    @pl.when(j < j_main)
    def _main():
        g = jax.lax.dot_general(x, wg_ref[0], dn,
                                preferred_element_type=jnp.float32)
        u = jax.lax.dot_general(x, wu_ref[0], dn,
                                preferred_element_type=jnp.float32)
        act = (g * jax.nn.sigmoid(g)) * u
        out_ref[...] += jax.lax.dot_general(act, w2_ref[...], dn,
                                            preferred_element_type=jnp.float32)

    @pl.when(j >= j_main)
    def _shared():
        s = jax.lax.dot_general(x, ws_ref[...], dn,
                                preferred_element_type=jnp.float32)
        act = s * jax.nn.sigmoid(s)
        glogit = jax.lax.dot_general(x, sgw_ref[...], dn,
                                     preferred_element_type=jnp.float32)
        gtok = jax.nn.sigmoid(glogit)
        out_ref[...] += gtok * jax.lax.dot_general(
            act, sw2_ref[...], dn, preferred_element_type=jnp.float32)


def kernel(hidden_states, w13, w2, gate, shared_w1, shared_w2, shared_gate_w):
    del gate
    bsz, seq_len, hidden = hidden_states.shape
    inter = shared_w1.shape[0]
    n_tokens = bsz * seq_len

    bt = min(256, n_tokens)
    bi = min(512, inter)
    assert n_tokens % bt == 0 and inter % bi == 0
    n_t = n_tokens // bt
    j_main = inter // bi
    j_total = 2 * j_main

    x = hidden_states.reshape(n_tokens, hidden)
    w13r = w13.reshape(2, inter, hidden)

    clamp_main = j_main - 1

    grid_spec = pl.GridSpec(
        grid=(n_t, j_total),
        in_specs=[
            pl.BlockSpec((bt, hidden), lambda t, j: (t, 0)),
            pl.BlockSpec((1, bi, hidden),
                         lambda t, j: (0, jnp.minimum(j, clamp_main), 0)),
            pl.BlockSpec((1, bi, hidden),
                         lambda t, j: (1, jnp.minimum(j, clamp_main), 0)),
            pl.BlockSpec((bi, hidden),
                         lambda t, j: (jnp.maximum(j - j_main, 0), 0)),
            pl.BlockSpec((hidden, bi),
                         lambda t, j: (0, jnp.minimum(j, clamp_main))),
            pl.BlockSpec((hidden, bi),
                         lambda t, j: (0, jnp.maximum(j - j_main, 0))),
            pl.BlockSpec((1, hidden), lambda t, j: (0, 0)),
        ],
        out_specs=pl.BlockSpec((bt, hidden), lambda t, j: (t, 0)),
    )

    out = pl.pallas_call(
        functools.partial(_ffn_body, j_main=j_main),
        grid_spec=grid_spec,
        out_shape=jax.ShapeDtypeStruct((n_tokens, hidden), jnp.float32),
        compiler_params=pltpu.CompilerParams(
            dimension_semantics=("parallel", "arbitrary"),
            vmem_limit_bytes=100 * 1024 * 1024,
        ),
    )(x, w13r, w13r, shared_w1, w2, shared_w2, shared_gate_w)

    return out.reshape(bsz, seq_len, hidden)


# speedup vs baseline: 1.7602x; 1.7286x over previous
import functools

import jax
import jax.numpy as jnp
from jax.experimental import pallas as pl
from jax.experimental.pallas import tpu as pltpu


def _ffn_body(x_ref, wg_ref, wu_ref, ws_ref, w2_ref, sw2_ref, sgw_ref,
              out_ref, *, j_main):
    j = pl.program_id(1)
    x = x_ref[...]

    @pl.when(j == 0)
    def _init():
        out_ref[...] = jnp.zeros_like(out_ref)

    dn = (((1,), (1,)), ((), ()))

    @pl.when(j < j_main)
    def _main():
        g = jax.lax.dot_general(x, wg_ref[0], dn,
                                preferred_element_type=jnp.float32)
        u = jax.lax.dot_general(x, wu_ref[0], dn,
                                preferred_element_type=jnp.float32)
        act = ((g * jax.nn.sigmoid(g)) * u).astype(jnp.bfloat16)
        out_ref[...] += jax.lax.dot_general(act, w2_ref[...], dn,
                                            preferred_element_type=jnp.float32)

    @pl.when(j >= j_main)
    def _shared():
        s = jax.lax.dot_general(x, ws_ref[...], dn,
                                preferred_element_type=jnp.float32)
        glogit = jax.lax.dot_general(x.astype(jnp.float32),
                                     sgw_ref[...].astype(jnp.float32), dn,
                                     preferred_element_type=jnp.float32)
        gtok = jax.nn.sigmoid(glogit)
        act = (s * jax.nn.sigmoid(s)).astype(jnp.bfloat16)
        out_ref[...] += gtok * jax.lax.dot_general(
            act, sw2_ref[...], dn, preferred_element_type=jnp.float32)


def kernel(hidden_states, w13, w2, gate, shared_w1, shared_w2, shared_gate_w):
    del gate
    bsz, seq_len, hidden = hidden_states.shape
    inter = shared_w1.shape[0]
    n_tokens = bsz * seq_len

    bt = min(1024, n_tokens)
    bi = min(512, inter)
    assert n_tokens % bt == 0 and inter % bi == 0
    n_t = n_tokens // bt
    j_main = inter // bi
    j_total = 2 * j_main

    x = hidden_states.reshape(n_tokens, hidden).astype(jnp.bfloat16)
    w13r = w13.reshape(2, inter, hidden).astype(jnp.bfloat16)
    sw1b = shared_w1.astype(jnp.bfloat16)
    w2b = w2.astype(jnp.bfloat16)
    sw2b = shared_w2.astype(jnp.bfloat16)
    sgwb = shared_gate_w.astype(jnp.bfloat16)

    clamp_main = j_main - 1

    grid_spec = pl.GridSpec(
        grid=(n_t, j_total),
        in_specs=[
            pl.BlockSpec((bt, hidden), lambda t, j: (t, 0)),
            pl.BlockSpec((1, bi, hidden),
                         lambda t, j: (0, jnp.minimum(j, clamp_main), 0)),
            pl.BlockSpec((1, bi, hidden),
                         lambda t, j: (1, jnp.minimum(j, clamp_main), 0)),
            pl.BlockSpec((bi, hidden),
                         lambda t, j: (jnp.maximum(j - j_main, 0), 0)),
            pl.BlockSpec((hidden, bi),
                         lambda t, j: (0, jnp.minimum(j, clamp_main))),
            pl.BlockSpec((hidden, bi),
                         lambda t, j: (0, jnp.maximum(j - j_main, 0))),
            pl.BlockSpec((1, hidden), lambda t, j: (0, 0)),
        ],
        out_specs=pl.BlockSpec((bt, hidden), lambda t, j: (t, 0)),
    )

    out = pl.pallas_call(
        functools.partial(_ffn_body, j_main=j_main),
        grid_spec=grid_spec,
        out_shape=jax.ShapeDtypeStruct((n_tokens, hidden), jnp.float32),
        compiler_params=pltpu.CompilerParams(
            dimension_semantics=("parallel", "arbitrary"),
            vmem_limit_bytes=63 * 1024 * 1024,
        ),
    )(x, w13r, w13r, sw1b, w2b, sw2b, sgwb)

    return out.reshape(bsz, seq_len, hidden)


# speedup vs baseline: 1.8102x; 1.0284x over previous
import functools

import jax
import jax.numpy as jnp
from jax.experimental import pallas as pl
from jax.experimental.pallas import tpu as pltpu


def _ffn_body(x_ref, wg_ref, wu_ref, ws_ref, w2_ref, sw2_ref, sgw_ref,
              out_ref, gtok_ref, *, j_main):
    j = pl.program_id(1)
    x = x_ref[...]

    @pl.when(j == 0)
    def _init():
        out_ref[...] = jnp.zeros_like(out_ref)
        glogit = jax.lax.dot_general(
            x.astype(jnp.float32), sgw_ref[...].astype(jnp.float32),
            (((1,), (1,)), ((), ())), preferred_element_type=jnp.float32)
        gtok_ref[...] = jax.nn.sigmoid(glogit)

    dn = (((1,), (1,)), ((), ()))

    @pl.when(j < j_main)
    def _main():
        g = jax.lax.dot_general(x, wg_ref[0], dn,
                                preferred_element_type=jnp.float32)
        u = jax.lax.dot_general(x, wu_ref[0], dn,
                                preferred_element_type=jnp.float32)
        act = ((g * jax.nn.sigmoid(g)) * u).astype(jnp.bfloat16)
        out_ref[...] += jax.lax.dot_general(act, w2_ref[...], dn,
                                            preferred_element_type=jnp.float32)

    @pl.when(j >= j_main)
    def _shared():
        s = jax.lax.dot_general(x, ws_ref[...], dn,
                                preferred_element_type=jnp.float32)
        act = ((s * jax.nn.sigmoid(s)) * gtok_ref[...]).astype(jnp.bfloat16)
        out_ref[...] += jax.lax.dot_general(act, sw2_ref[...], dn,
                                            preferred_element_type=jnp.float32)


def kernel(hidden_states, w13, w2, gate, shared_w1, shared_w2, shared_gate_w):
    del gate
    bsz, seq_len, hidden = hidden_states.shape
    inter = shared_w1.shape[0]
    n_tokens = bsz * seq_len

    bt = min(1024, n_tokens)
    bi = min(512, inter)
    assert n_tokens % bt == 0 and inter % bi == 0
    n_t = n_tokens // bt
    j_main = inter // bi
    j_total = 2 * j_main

    x = hidden_states.reshape(n_tokens, hidden).astype(jnp.bfloat16)
    w13r = w13.reshape(2, inter, hidden).astype(jnp.bfloat16)
    sw1b = shared_w1.astype(jnp.bfloat16)
    w2b = w2.astype(jnp.bfloat16)
    sw2b = shared_w2.astype(jnp.bfloat16)
    sgwb = shared_gate_w.astype(jnp.bfloat16)

    clamp_main = j_main - 1

    grid_spec = pltpu.PrefetchScalarGridSpec(
        num_scalar_prefetch=0,
        grid=(n_t, j_total),
        in_specs=[
            pl.BlockSpec((bt, hidden), lambda t, j: (t, 0)),
            pl.BlockSpec((1, bi, hidden),
                         lambda t, j: (0, jnp.minimum(j, clamp_main), 0)),
            pl.BlockSpec((1, bi, hidden),
                         lambda t, j: (1, jnp.minimum(j, clamp_main), 0)),
            pl.BlockSpec((bi, hidden),
                         lambda t, j: (jnp.maximum(j - j_main, 0), 0)),
            pl.BlockSpec((hidden, bi),
                         lambda t, j: (0, jnp.minimum(j, clamp_main))),
            pl.BlockSpec((hidden, bi),
                         lambda t, j: (0, jnp.maximum(j - j_main, 0))),
            pl.BlockSpec((1, hidden), lambda t, j: (0, 0)),
        ],
        out_specs=pl.BlockSpec((bt, hidden), lambda t, j: (t, 0)),
        scratch_shapes=[pltpu.VMEM((bt, 1), jnp.float32)],
    )

    out = pl.pallas_call(
        functools.partial(_ffn_body, j_main=j_main),
        grid_spec=grid_spec,
        out_shape=jax.ShapeDtypeStruct((n_tokens, hidden), jnp.float32),
        compiler_params=pltpu.CompilerParams(
            dimension_semantics=("parallel", "arbitrary"),
            vmem_limit_bytes=63 * 1024 * 1024,
        ),
    )(x, w13r, w13r, sw1b, w2b, sw2b, sgwb)

    return out.reshape(bsz, seq_len, hidden)
